# Initial kernel scaffold; baseline (speedup 1.0000x reference)
import jax
import jax.numpy as jnp
from jax.experimental import pallas as pl


def kernel(input, weight):
    raise NotImplementedError("write your pallas kernel here")


# speedup vs baseline: 1.8432x; 1.8432x over previous
import functools

import jax
import jax.numpy as jnp
from jax import lax
from jax.experimental import pallas as pl
from jax.experimental.pallas import tpu as pltpu
from jax.experimental.pallas import tpu_sc as plsc

_VOCAB = 1000000
_EMBED = 64
_BATCH = 16384
_HIST = 50
_NTOT = _BATCH * _HIST
_NW = 32
_RPW = _NTOT // _NW
_GW = 128
_C = 1024
_KG = _C // _GW
_NCH = _RPW // _C


def _sc_gather(idx_hbm, table_hbm, out_hbm, idx_v, rows_v, gsem, osem):
    wid = lax.axis_index("s") * 2 + lax.axis_index("c")
    base = wid * _RPW

    def chunk(g, carry):
        row0 = pl.multiple_of(base + g * _C, _C)
        pltpu.sync_copy(idx_hbm.at[pl.ds(pl.multiple_of(row0 // _GW, _KG), _KG)], idx_v)
        copies = [
            pltpu.async_copy(
                table_hbm.at[idx_v.at[j]],
                rows_v.at[pl.ds(j * _GW, _GW)],
                gsem,
            )
            for j in range(_KG)
        ]
        for cp in copies:
            cp.wait()
        pltpu.async_copy(rows_v, out_hbm.at[pl.ds(row0, _C)], osem).wait()
        return carry

    lax.fori_loop(0, _NCH, chunk, 0)


_mesh = plsc.VectorSubcoreMesh(core_axis_name="c", subcore_axis_name="s")

_gather_call = functools.partial(
    pl.kernel,
    out_type=jax.ShapeDtypeStruct((_NTOT, _EMBED), jnp.float32),
    mesh=_mesh,
    compiler_params=pltpu.CompilerParams(use_tc_tiling_on_sc=False),
    scratch_types=[
        pltpu.VMEM((_KG, _GW), jnp.int32),
        pltpu.VMEM((_C, _EMBED), jnp.float32),
        pltpu.SemaphoreType.DMA,
        pltpu.SemaphoreType.DMA,
    ],
)(_sc_gather)


@jax.jit
def kernel(input, weight):
    idx = input.reshape(_NTOT // _GW, _GW).astype(jnp.int32)
    rows = _gather_call(idx, weight)
    return rows.reshape(_BATCH, _HIST, _EMBED)


# speedup vs baseline: 1.8736x; 1.0165x over previous
import functools

import jax
import jax.numpy as jnp
from jax import lax
from jax.experimental import pallas as pl
from jax.experimental.pallas import tpu as pltpu
from jax.experimental.pallas import tpu_sc as plsc

_VOCAB = 1000000
_EMBED = 64
_BATCH = 16384
_HIST = 50
_NTOT = _BATCH * _HIST
_NW = 32
_RPW = _NTOT // _NW
_GW = 128
_C = 512
_KG = _C // _GW
_NCH = _RPW // _C
_IDXROWS = _RPW // _GW


def _sc_gather(idx_hbm, table_hbm, out_hbm, idx_v, rows_a, rows_b, gsem_a,
               gsem_b, osem_a, osem_b):
    wid = lax.axis_index("s") * 2 + lax.axis_index("c")
    base = wid * _RPW

    pltpu.sync_copy(
        idx_hbm.at[pl.ds(pl.multiple_of(wid * _IDXROWS, 8), _IDXROWS)], idx_v)

    def fire_gathers(g, rows_v, sem):
        return [
            pltpu.async_copy(
                table_hbm.at[idx_v.at[g * _KG + j]],
                rows_v.at[pl.ds(j * _GW, _GW)],
                sem,
            )
            for j in range(_KG)
        ]

    def drain_gathers(rows_v, sem):
        for j in range(_KG):
            pltpu.make_async_copy(
                table_hbm.at[idx_v.at[j]],
                rows_v.at[pl.ds(j * _GW, _GW)],
                sem,
            ).wait()

    def store(g, rows_v, sem):
        return pltpu.async_copy(
            rows_v, out_hbm.at[pl.ds(pl.multiple_of(base + g * _C, _C), _C)],
            sem)

    def wait_store(g, rows_v, sem):
        pltpu.make_async_copy(
            rows_v, out_hbm.at[pl.ds(pl.multiple_of(base + g * _C, _C), _C)],
            sem).wait()

    fire_gathers(0, rows_a, gsem_a)
    drain_gathers(rows_a, gsem_a)
    fire_gathers(1, rows_b, gsem_b)
    store(0, rows_a, osem_a)

    def body(p, carry):
        g = 2 * p + 1
        drain_gathers(rows_b, gsem_b)
        wait_store(g - 1, rows_a, osem_a)
        fire_gathers(g + 1, rows_a, gsem_a)
        store(g, rows_b, osem_b)
        drain_gathers(rows_a, gsem_a)
        wait_store(g, rows_b, osem_b)
        fire_gathers(g + 2, rows_b, gsem_b)
        store(g + 1, rows_a, osem_a)
        return carry

    lax.fori_loop(0, _NCH // 2 - 1, body, 0)

    drain_gathers(rows_b, gsem_b)
    store(_NCH - 1, rows_b, osem_b)
    wait_store(_NCH - 2, rows_a, osem_a)
    wait_store(_NCH - 1, rows_b, osem_b)


_mesh = plsc.VectorSubcoreMesh(core_axis_name="c", subcore_axis_name="s")

_gather_call = functools.partial(
    pl.kernel,
    out_type=jax.ShapeDtypeStruct((_NTOT, _EMBED), jnp.float32),
    mesh=_mesh,
    compiler_params=pltpu.CompilerParams(use_tc_tiling_on_sc=False),
    scratch_types=[
        pltpu.VMEM((_IDXROWS, _GW), jnp.int32),
        pltpu.VMEM((_C, _EMBED), jnp.float32),
        pltpu.VMEM((_C, _EMBED), jnp.float32),
        pltpu.SemaphoreType.DMA,
        pltpu.SemaphoreType.DMA,
        pltpu.SemaphoreType.DMA,
        pltpu.SemaphoreType.DMA,
    ],
)(_sc_gather)


@jax.jit
def kernel(input, weight):
    idx = input.reshape(_NTOT // _GW, _GW).astype(jnp.int32)
    rows = _gather_call(idx, weight)
    return rows.reshape(_BATCH, _HIST, _EMBED)
